# baseline (speedup 1.0000x reference)
import functools

import jax
import jax.numpy as jnp
from jax import lax
from jax.experimental import pallas as pl
from jax.experimental.pallas import tpu as pltpu
from jax.experimental.pallas import tpu_sc as plsc

_NE = 8192
_D = 256
_B = 8192
_BI = 1024
_BJ = 8192
_I = _B // _BI
_J = _NE // _BJ
_CHUNK = 128


def _argmin_body(x_ref, et_ref, hn_ref, en_ref, idx_ref, val_ref):
    j = pl.program_id(1)
    x = x_ref[...]
    et = (2.0 * et_ref[...]).T
    hn = hn_ref[...]
    en = en_ref[...].reshape(1, _BJ)
    s = (hn + en) - jnp.dot(x, et, preferred_element_type=jnp.float32)
    m = jnp.min(s, axis=1, keepdims=True)
    _ONE = jnp.int32(0x3F800000)
    iota_f = lax.bitcast_convert_type(
        lax.broadcasted_iota(jnp.int32, (_BI, _BJ), 1) + _ONE, jnp.float32)
    sentinel = lax.bitcast_convert_type(jnp.int32(_NE) + _ONE, jnp.float32)
    loc_f = jnp.min(jnp.where(s == m, iota_f, sentinel),
                    axis=1, keepdims=True)
    loc = lax.bitcast_convert_type(loc_f, jnp.int32) - _ONE
    gidx = loc + j * _BJ

    @pl.when(j == 0)
    def _():
        val_ref[...] = m
        idx_ref[...] = gidx

    @pl.when(j > 0)
    def _():
        prev = val_ref[...]
        better = m < prev
        val_ref[...] = jnp.where(better, m, prev)
        idx_ref[...] = jnp.where(better, gidx, idx_ref[...])


def _tc_argmin(x, embT, hn, en3):
    return pl.pallas_call(
        _argmin_body,
        grid=(_I, _J),
        in_specs=[
            pl.BlockSpec((_BI, _D), lambda i, j: (i, 0)),
            pl.BlockSpec((_BJ, _D), lambda i, j: (j, 0)),
            pl.BlockSpec((_BI, 1), lambda i, j: (i, 0)),
            pl.BlockSpec((1, 1, _BJ), lambda i, j: (j, 0, 0)),
        ],
        out_specs=pl.BlockSpec((_BI, 1), lambda i, j: (i, 0)),
        out_shape=jax.ShapeDtypeStruct((_B, 1), jnp.int32),
        scratch_shapes=[pltpu.VMEM((_BI, 1), jnp.float32)],
    )(x, embT, hn, en3)


@functools.cache
def _make_sc_gather():
    info = plsc.get_sparse_core_info()
    nc, ns = info.num_cores, info.num_subcores
    nw = nc * ns
    bpw = _B // nw
    chunks = bpw // _CHUNK
    mesh = plsc.VectorSubcoreMesh(core_axis_name="c", subcore_axis_name="s")

    @functools.partial(
        pl.kernel,
        mesh=mesh,
        out_type=jax.ShapeDtypeStruct((_B, _D), jnp.float32),
        scratch_types=[
            pltpu.VMEM((chunks, _CHUNK), jnp.int32),
            pltpu.VMEM((bpw, _D), jnp.float32),
            pltpu.SemaphoreType.DMA,
            pltpu.SemaphoreType.DMA,
        ],
    )
    def gather(table_hbm, idx_hbm, out_hbm, idx_v, rows_v, sem, sem_out):
        wid = lax.axis_index("s") * nc + lax.axis_index("c")
        pltpu.sync_copy(idx_hbm.at[pl.ds(wid * chunks, chunks)], idx_v)
        gathers = []
        for c in range(chunks):
            gathers.append(pltpu.async_copy(
                table_hbm.at[idx_v.at[c]],
                rows_v.at[pl.ds(c * _CHUNK, _CHUNK)], sem))
        scatters = []
        for c in range(chunks):
            gathers[c].wait()
            scatters.append(pltpu.async_copy(
                rows_v.at[pl.ds(c * _CHUNK, _CHUNK)],
                out_hbm.at[pl.ds(wid * bpw + c * _CHUNK, _CHUNK)], sem_out))
        for cp in scatters:
            cp.wait()

    return gather


def kernel(hidden_states, emb_weights):
    x = hidden_states.reshape((_B, _D))
    hn = jnp.sum(x ** 2, axis=1, keepdims=True)
    en3 = jnp.sum(emb_weights ** 2, axis=1).reshape(_J, 1, _BJ)
    idx2d = _tc_argmin(x, emb_weights, hn, en3)
    idx = idx2d.reshape((_B,))
    idx_rows = idx.reshape((_B // _CHUNK, _CHUNK))
    z_q = _make_sc_gather()(emb_weights, idx_rows).reshape(hidden_states.shape)
    return z_q, idx.reshape(hidden_states.shape[0], -1)


# speedup vs baseline: 1.0211x; 1.0211x over previous
import functools

import jax
import jax.numpy as jnp
from jax import lax
from jax.experimental import pallas as pl
from jax.experimental.pallas import tpu as pltpu
from jax.experimental.pallas import tpu_sc as plsc

_NE = 8192
_D = 256
_B = 8192
_BI = 2048
_BJ = 4096
_I = _B // _BI
_J = _NE // _BJ
_CHUNK = 128


def _argmin_body(x_ref, et_ref, hn_ref, idx_ref, val_ref):
    j = pl.program_id(1)
    x = x_ref[...]
    et = (2.0 * et_ref[...]).T
    hn = hn_ref[...]
    en = 0.25 * jnp.sum(et * et, axis=0, keepdims=True)
    s = (hn + en) - jnp.dot(x, et, preferred_element_type=jnp.float32)
    m = jnp.min(s, axis=1, keepdims=True)
    _ONE = jnp.int32(0x3F800000)
    iota_f = lax.bitcast_convert_type(
        lax.broadcasted_iota(jnp.int32, (_BI, _BJ), 1) + _ONE, jnp.float32)
    sentinel = lax.bitcast_convert_type(jnp.int32(_NE) + _ONE, jnp.float32)
    loc_f = jnp.min(jnp.where(s == m, iota_f, sentinel),
                    axis=1, keepdims=True)
    loc = lax.bitcast_convert_type(loc_f, jnp.int32) - _ONE
    gidx = loc + j * _BJ

    @pl.when(j == 0)
    def _():
        val_ref[...] = m
        idx_ref[...] = gidx

    @pl.when(j > 0)
    def _():
        prev = val_ref[...]
        better = m < prev
        val_ref[...] = jnp.where(better, m, prev)
        idx_ref[...] = jnp.where(better, gidx, idx_ref[...])


def _tc_argmin(x, embT, hn):
    return pl.pallas_call(
        _argmin_body,
        grid=(_I, _J),
        in_specs=[
            pl.BlockSpec((_BI, _D), lambda i, j: (i, 0)),
            pl.BlockSpec((_BJ, _D), lambda i, j: (j, 0)),
            pl.BlockSpec((_BI, 1), lambda i, j: (i, 0)),
        ],
        out_specs=pl.BlockSpec((_BI, 1), lambda i, j: (i, 0)),
        out_shape=jax.ShapeDtypeStruct((_B, 1), jnp.int32),
        scratch_shapes=[pltpu.VMEM((_BI, 1), jnp.float32)],
    )(x, embT, hn)


@functools.cache
def _make_sc_gather():
    info = plsc.get_sparse_core_info()
    nc, ns = info.num_cores, info.num_subcores
    nw = nc * ns
    bpw = _B // nw
    chunks = bpw // _CHUNK
    mesh = plsc.VectorSubcoreMesh(core_axis_name="c", subcore_axis_name="s")

    @functools.partial(
        pl.kernel,
        mesh=mesh,
        out_type=jax.ShapeDtypeStruct((_B, _D), jnp.float32),
        scratch_types=[
            pltpu.VMEM((chunks, _CHUNK), jnp.int32),
            pltpu.VMEM((bpw, _D), jnp.float32),
            pltpu.SemaphoreType.DMA,
            pltpu.SemaphoreType.DMA,
        ],
    )
    def gather(table_hbm, idx_hbm, out_hbm, idx_v, rows_v, sem, sem_out):
        wid = lax.axis_index("s") * nc + lax.axis_index("c")
        pltpu.sync_copy(idx_hbm.at[pl.ds(wid * chunks, chunks)], idx_v)
        gathers = []
        for c in range(chunks):
            gathers.append(pltpu.async_copy(
                table_hbm.at[idx_v.at[c]],
                rows_v.at[pl.ds(c * _CHUNK, _CHUNK)], sem))
        scatters = []
        for c in range(chunks):
            gathers[c].wait()
            scatters.append(pltpu.async_copy(
                rows_v.at[pl.ds(c * _CHUNK, _CHUNK)],
                out_hbm.at[pl.ds(wid * bpw + c * _CHUNK, _CHUNK)], sem_out))
        for cp in scatters:
            cp.wait()

    return gather


def kernel(hidden_states, emb_weights):
    x = hidden_states.reshape((_B, _D))
    hn = jnp.sum(x ** 2, axis=1, keepdims=True)
    idx2d = _tc_argmin(x, emb_weights, hn)
    idx = idx2d.reshape((_B,))
    idx_rows = idx.reshape((_B // _CHUNK, _CHUNK))
    z_q = _make_sc_gather()(emb_weights, idx_rows).reshape(hidden_states.shape)
    return z_q, idx.reshape(hidden_states.shape[0], -1)
